# baseline (speedup 1.0000x reference)
import jax
import jax.numpy as jnp
from jax.experimental import pallas as pl

_V_TILE = 107
_P_CHUNK = 3


def _head_body(xt_ref, xf_ref, wt_ref, wf_ref, b_ref, o_ref):
    p = pl.program_id(1)
    vt, pc, bb, d = xt_ref.shape
    mb = vt * bb
    tw = o_ref.shape[1]

    dn = (((1,), (0,)), ((), ()))
    acc = jax.lax.dot_general(
        xt_ref[:, 0, :, :].reshape(mb, d), wt_ref[0], dn,
        preferred_element_type=jnp.float32)
    acc += jax.lax.dot_general(
        xf_ref[:, 0, :, :].reshape(mb, d), wf_ref[0], dn,
        preferred_element_type=jnp.float32)
    for q in range(1, pc):
        acc += jax.lax.dot_general(
            xt_ref[:, q, :, :].reshape(mb, d), wt_ref[q], dn,
            preferred_element_type=jnp.float32)
        acc += jax.lax.dot_general(
            xf_ref[:, q, :, :].reshape(mb, d), wf_ref[q], dn,
            preferred_element_type=jnp.float32)

    @pl.when(p == 0)
    def _init():
        o_ref[...] = acc + b_ref[...]

    @pl.when(p != 0)
    def _accum():
        o_ref[...] += acc


def kernel(x_time, x_frequency, W, b):
    B, V, D, P = x_time.shape
    K = D * P
    TW = W.shape[0]

    xt = jnp.transpose(x_time, (1, 3, 0, 2))
    xf = jnp.transpose(x_frequency, (1, 3, 0, 2))

    Wt = W[:, :K].reshape(TW, D, P).transpose(2, 1, 0)
    Wf = W[:, K:].reshape(TW, D, P).transpose(2, 1, 0)
    b2 = b.reshape(1, TW)

    grid = (V // _V_TILE, P // _P_CHUNK)
    out = pl.pallas_call(
        _head_body,
        grid=grid,
        in_specs=[
            pl.BlockSpec((_V_TILE, _P_CHUNK, B, D), lambda i, p: (i, p, 0, 0)),
            pl.BlockSpec((_V_TILE, _P_CHUNK, B, D), lambda i, p: (i, p, 0, 0)),
            pl.BlockSpec((_P_CHUNK, D, TW), lambda i, p: (p, 0, 0)),
            pl.BlockSpec((_P_CHUNK, D, TW), lambda i, p: (p, 0, 0)),
            pl.BlockSpec((1, TW), lambda i, p: (0, 0)),
        ],
        out_specs=pl.BlockSpec((_V_TILE * B, TW), lambda i, p: (i, 0)),
        out_shape=jax.ShapeDtypeStruct((V * B, TW), jnp.float32),
    )(xt, xf, Wt, Wf, b2)

    return out.reshape(V, B, TW).transpose(1, 0, 2)


# speedup vs baseline: 1.0107x; 1.0107x over previous
import jax
import jax.numpy as jnp
from jax.experimental import pallas as pl

_V_TILE = 107
_P_CHUNK = 2


def _head_body(xt_ref, xf_ref, wt_ref, wf_ref, b_ref, o_ref):
    p = pl.program_id(1)
    vt, pc, bb, d = xt_ref.shape
    mb = vt * bb
    tw = o_ref.shape[1]

    dn = (((1,), (0,)), ((), ()))
    acc = jax.lax.dot_general(
        xt_ref[:, 0, :, :].reshape(mb, d), wt_ref[0], dn,
        preferred_element_type=jnp.float32)
    acc += jax.lax.dot_general(
        xf_ref[:, 0, :, :].reshape(mb, d), wf_ref[0], dn,
        preferred_element_type=jnp.float32)
    for q in range(1, pc):
        acc += jax.lax.dot_general(
            xt_ref[:, q, :, :].reshape(mb, d), wt_ref[q], dn,
            preferred_element_type=jnp.float32)
        acc += jax.lax.dot_general(
            xf_ref[:, q, :, :].reshape(mb, d), wf_ref[q], dn,
            preferred_element_type=jnp.float32)

    @pl.when(p == 0)
    def _init():
        o_ref[...] = acc + b_ref[...]

    @pl.when(p != 0)
    def _accum():
        o_ref[...] += acc


def kernel(x_time, x_frequency, W, b):
    B, V, D, P = x_time.shape
    K = D * P
    TW = W.shape[0]

    xt = jnp.transpose(x_time, (1, 3, 0, 2))
    xf = jnp.transpose(x_frequency, (1, 3, 0, 2))

    Wt = W[:, :K].reshape(TW, D, P).transpose(2, 1, 0)
    Wf = W[:, K:].reshape(TW, D, P).transpose(2, 1, 0)
    b2 = b.reshape(1, TW)

    grid = (V // _V_TILE, P // _P_CHUNK)
    out = pl.pallas_call(
        _head_body,
        grid=grid,
        in_specs=[
            pl.BlockSpec((_V_TILE, _P_CHUNK, B, D), lambda i, p: (i, p, 0, 0)),
            pl.BlockSpec((_V_TILE, _P_CHUNK, B, D), lambda i, p: (i, p, 0, 0)),
            pl.BlockSpec((_P_CHUNK, D, TW), lambda i, p: (p, 0, 0)),
            pl.BlockSpec((_P_CHUNK, D, TW), lambda i, p: (p, 0, 0)),
            pl.BlockSpec((1, TW), lambda i, p: (0, 0)),
        ],
        out_specs=pl.BlockSpec((_V_TILE * B, TW), lambda i, p: (i, 0)),
        out_shape=jax.ShapeDtypeStruct((V * B, TW), jnp.float32),
    )(xt, xf, Wt, Wf, b2)

    return out.reshape(V, B, TW).transpose(1, 0, 2)


# speedup vs baseline: 1.2242x; 1.2112x over previous
import jax
import jax.numpy as jnp
from jax.experimental import pallas as pl
from jax.experimental.pallas import tpu as pltpu

_V_TILE = 107
_P_CHUNK = 2


def _head_body(xtl_ref, xth_ref, xfl_ref, xfh_ref, wt_ref, wf_ref, b_ref,
               o_ref, acc_ref):
    i = pl.program_id(0)
    p = pl.program_id(1)
    np_steps = pl.num_programs(1)
    vt, pc, bh, d = xtl_ref.shape
    mh = vt * bh
    tw = acc_ref.shape[2]
    bb = 2 * bh

    dn = (((1,), (0,)), ((), ()))
    accs = []
    for half_refs in ((xtl_ref, xfl_ref), (xth_ref, xfh_ref)):
        xt_h, xf_h = half_refs
        a = jax.lax.dot_general(
            xt_h[:, 0, :, :].reshape(mh, d), wt_ref[0], dn,
            preferred_element_type=jnp.float32)
        a += jax.lax.dot_general(
            xf_h[:, 0, :, :].reshape(mh, d), wf_ref[0], dn,
            preferred_element_type=jnp.float32)
        for q in range(1, pc):
            a += jax.lax.dot_general(
                xt_h[:, q, :, :].reshape(mh, d), wt_ref[q], dn,
                preferred_element_type=jnp.float32)
            a += jax.lax.dot_general(
                xf_h[:, q, :, :].reshape(mh, d), wf_ref[q], dn,
                preferred_element_type=jnp.float32)
        accs.append(a.reshape(vt, bh, tw))

    @pl.when(p == 0)
    def _init():
        acc_ref[:, :bh, :] = accs[0] + b_ref[...]
        acc_ref[:, bh:, :] = accs[1] + b_ref[...]

    @pl.when(p != 0)
    def _accum():
        acc_ref[:, :bh, :] += accs[0]
        acc_ref[:, bh:, :] += accs[1]

    @pl.when(p == np_steps - 1)
    def _flush():
        t = acc_ref[...].transpose(1, 2, 0)
        for iv in range(3):
            @pl.when(i == iv)
            def _store(iv=iv):
                o_ref[:, :, iv * vt:(iv + 1) * vt] = t


def kernel(x_time, x_frequency, W, b):
    B, V, D, P = x_time.shape
    K = D * P
    TW = W.shape[0]

    xt = jnp.transpose(x_time, (1, 3, 0, 2))
    xf = jnp.transpose(x_frequency, (1, 3, 0, 2))

    Wt = W[:, :K].reshape(TW, D, P).transpose(2, 1, 0)
    Wf = W[:, K:].reshape(TW, D, P).transpose(2, 1, 0)
    b2 = b.reshape(1, TW)

    grid = (V // _V_TILE, P // _P_CHUNK)
    out = pl.pallas_call(
        _head_body,
        grid=grid,
        in_specs=[
            pl.BlockSpec((_V_TILE, _P_CHUNK, B // 2, D), lambda i, p: (i, p, 0, 0)),
            pl.BlockSpec((_V_TILE, _P_CHUNK, B // 2, D), lambda i, p: (i, p, 1, 0)),
            pl.BlockSpec((_V_TILE, _P_CHUNK, B // 2, D), lambda i, p: (i, p, 0, 0)),
            pl.BlockSpec((_V_TILE, _P_CHUNK, B // 2, D), lambda i, p: (i, p, 1, 0)),
            pl.BlockSpec((_P_CHUNK, D, TW), lambda i, p: (p, 0, 0)),
            pl.BlockSpec((_P_CHUNK, D, TW), lambda i, p: (p, 0, 0)),
            pl.BlockSpec((1, TW), lambda i, p: (0, 0)),
        ],
        out_specs=pl.BlockSpec((B, TW, V), lambda i, p: (0, 0, 0)),
        out_shape=jax.ShapeDtypeStruct((B, TW, V), jnp.float32),
        scratch_shapes=[pltpu.VMEM((_V_TILE, B, TW), jnp.float32)],
    )(xt, xt, xf, xf, Wt, Wf, b2)

    return jnp.transpose(out, (0, 2, 1))


# speedup vs baseline: 1.2266x; 1.0020x over previous
import jax
import jax.numpy as jnp
from jax.experimental import pallas as pl
from jax.experimental.pallas import tpu as pltpu

_V_TILE = 107
_P_CHUNK = 2


def _head_body(xt_ref, xf_ref, wt_ref, wf_ref, b_ref, o_ref, acc_ref):
    i = pl.program_id(0)
    p = pl.program_id(1)
    np_steps = pl.num_programs(1)
    vt, pc, bb, d = xt_ref.shape
    mb = vt * bb
    tw = acc_ref.shape[1]

    dn = (((1,), (0,)), ((), ()))
    acc = jax.lax.dot_general(
        xt_ref[:, 0, :, :].reshape(mb, d), wt_ref[0], dn,
        preferred_element_type=jnp.float32)
    acc += jax.lax.dot_general(
        xf_ref[:, 0, :, :].reshape(mb, d), wf_ref[0], dn,
        preferred_element_type=jnp.float32)
    for q in range(1, pc):
        acc += jax.lax.dot_general(
            xt_ref[:, q, :, :].reshape(mb, d), wt_ref[q], dn,
            preferred_element_type=jnp.float32)
        acc += jax.lax.dot_general(
            xf_ref[:, q, :, :].reshape(mb, d), wf_ref[q], dn,
            preferred_element_type=jnp.float32)

    @pl.when(p == 0)
    def _init():
        acc_ref[...] = acc + b_ref[...]

    @pl.when(p != 0)
    def _accum():
        acc_ref[...] += acc

    @pl.when(p == np_steps - 1)
    def _flush():
        t = acc_ref[...].reshape(vt, bb, tw).transpose(1, 2, 0)
        for iv in range(o_ref.shape[2] // vt):
            @pl.when(i == iv)
            def _store(iv=iv):
                o_ref[:, :, iv * vt:(iv + 1) * vt] = t


def kernel(x_time, x_frequency, W, b):
    B, V, D, P = x_time.shape
    K = D * P
    TW = W.shape[0]

    xt = jnp.transpose(x_time, (1, 3, 0, 2))
    xf = jnp.transpose(x_frequency, (1, 3, 0, 2))

    Wt = W[:, :K].reshape(TW, D, P).transpose(2, 1, 0)
    Wf = W[:, K:].reshape(TW, D, P).transpose(2, 1, 0)
    b2 = b.reshape(1, TW)

    grid = (V // _V_TILE, P // _P_CHUNK)
    out = pl.pallas_call(
        _head_body,
        grid=grid,
        in_specs=[
            pl.BlockSpec((_V_TILE, _P_CHUNK, B, D), lambda i, p: (i, p, 0, 0)),
            pl.BlockSpec((_V_TILE, _P_CHUNK, B, D), lambda i, p: (i, p, 0, 0)),
            pl.BlockSpec((_P_CHUNK, D, TW), lambda i, p: (p, 0, 0)),
            pl.BlockSpec((_P_CHUNK, D, TW), lambda i, p: (p, 0, 0)),
            pl.BlockSpec((1, TW), lambda i, p: (0, 0)),
        ],
        out_specs=pl.BlockSpec((B, TW, V), lambda i, p: (0, 0, 0)),
        out_shape=jax.ShapeDtypeStruct((B, TW, V), jnp.float32),
        scratch_shapes=[pltpu.VMEM((_V_TILE * B, TW), jnp.float32)],
    )(xt, xf, Wt, Wf, b2)

    return jnp.transpose(out, (0, 2, 1))
